# baseline (speedup 1.0000x reference)
import functools

import jax
import jax.numpy as jnp
from jax import lax
from jax.experimental import pallas as pl
from jax.experimental.pallas import tpu as pltpu
from jax.experimental.pallas import tpu_sc as plsc

E = 16
D = 1024
DFF = 1024
N = 8192
K = N // E
L = 16
NC, NS = 2, 16
NW = NC * NS

_MESH = dict(core_axis_name="c", subcore_axis_name="s", num_cores=NC,
             num_subcores=NS)


_TB = 1024


def _router_body(x_ref, w_ref, b_ref, out_ref):
    lg = lax.dot_general(w_ref[...], x_ref[...],
                         (((0,), (1,)), ((), ())),
                         preferred_element_type=jnp.float32)
    lg = lg + b_ref[...]
    m = jnp.max(lg, axis=0, keepdims=True)
    ex = jnp.exp(lg - m)
    out_ref[...] = ex / jnp.sum(ex, axis=0, keepdims=True)


def _router(xf, w, b2d):
    return pl.pallas_call(
        _router_body,
        grid=(N // _TB,),
        in_specs=[
            pl.BlockSpec((_TB, D), lambda i: (i, 0)),
            pl.BlockSpec((D, E), lambda i: (0, 0)),
            pl.BlockSpec((E, 1), lambda i: (0, 0)),
        ],
        out_specs=pl.BlockSpec((E, _TB), lambda i: (0, i)),
        out_shape=jax.ShapeDtypeStruct((E, N), jnp.float32),
    )(xf, w, b2d)


def _thresh_body(pt_ref, tau_ref, bud_ref):
    bits = lax.bitcast_convert_type(pt_ref[...], jnp.int32)

    def body(_, carry):
        lo, hi = carry
        mid = lo + lax.div(hi - lo + 1, 2)
        cnt = jnp.sum((bits >= mid).astype(jnp.int32), axis=1, keepdims=True)
        ge = cnt >= K
        return jnp.where(ge, mid, lo), jnp.where(ge, hi, mid - 1)

    lo0 = jnp.zeros((E, 1), jnp.int32)
    hi0 = jnp.full((E, 1), jnp.int32(0x3F800001))
    lo, _ = lax.fori_loop(0, 31, body, (lo0, hi0))
    cnt_gt = jnp.sum((bits > lo).astype(jnp.int32), axis=1, keepdims=True)
    tau_ref[...] = jnp.broadcast_to(lo, (E, 128))
    bud_ref[...] = jnp.broadcast_to(K - cnt_gt, (E, 128))


def _thresh(probs_t):
    return pl.pallas_call(
        _thresh_body,
        out_shape=(jax.ShapeDtypeStruct((E, 128), jnp.int32),
                   jax.ShapeDtypeStruct((E, 128), jnp.int32)),
    )(probs_t)


@functools.partial(
    pl.kernel,
    out_type=(jax.ShapeDtypeStruct((E, K), jnp.int32),
              jax.ShapeDtypeStruct((E, K), jnp.float32)),
    mesh=plsc.VectorSubcoreMesh(**_MESH),
    compiler_params=pltpu.CompilerParams(needs_layout_passes=False),
    scratch_types=[
        pltpu.VMEM((N,), jnp.float32),
        pltpu.VMEM((K,), jnp.int32),
        pltpu.VMEM((K,), jnp.float32),
        pltpu.VMEM((L,), jnp.int32),
        pltpu.VMEM((L,), jnp.int32),
    ],
)
def _compact(pt_hbm, tau_hbm, bud_hbm, routes_hbm, vals_hbm,
             row_v, idx_v, val_v, tau_v, bud_v):
    wid = lax.axis_index("s") * NC + lax.axis_index("c")

    @pl.when(wid < E)
    def _():
        e = wid
        pltpu.sync_copy(pt_hbm.at[e], row_v)
        pltpu.sync_copy(tau_hbm.at[e, pl.ds(0, L)], tau_v)
        pltpu.sync_copy(bud_hbm.at[e, pl.ds(0, L)], bud_v)
        lane = lax.iota(jnp.int32, L)
        tau_vec = tau_v[...]
        bud_vec0 = bud_v[...]

        def body(j, carry):
            ptr, bud = carry
            p16 = row_v[pl.ds(j * L, L)]
            bits = plsc.bitcast(p16, jnp.int32)
            gt = bits > tau_vec
            eq = bits == tau_vec
            eqc = plsc.cumsum(jnp.where(eq, 1, 0))
            take_eq = eq & (eqc <= bud)
            mask = gt | take_eq
            pos = ptr + plsc.cumsum(jnp.where(mask, 1, 0)) - 1
            ids = j * L + lane
            plsc.store_scatter(idx_v, [pos], ids, mask=mask)
            plsc.store_scatter(val_v, [pos], p16, mask=mask)
            cnt = plsc.all_reduce_population_count(mask)
            ceq = plsc.all_reduce_population_count(take_eq)
            return ptr + cnt, bud - ceq

        zero16 = jnp.zeros((L,), jnp.int32)
        lax.fori_loop(0, N // L, body, (zero16, bud_vec0))
        pltpu.sync_copy(idx_v, routes_hbm.at[e])
        pltpu.sync_copy(val_v, vals_hbm.at[e])


_GC = 32


def _make_gather(rows):
    bpw = rows // NW

    @functools.partial(
        pl.kernel,
        out_type=jax.ShapeDtypeStruct((rows, D), jnp.float32),
        mesh=plsc.VectorSubcoreMesh(**_MESH),
        compiler_params=pltpu.CompilerParams(needs_layout_passes=False),
        scratch_types=[
            pltpu.VMEM((bpw,), jnp.int32),
            pltpu.VMEM((2, _GC, D), jnp.float32),
            pltpu.SemaphoreType.DMA,
            pltpu.SemaphoreType.DMA,
        ],
    )
    def g(xf_hbm, ridx_hbm, xg_hbm, idx_v, rows_v, sem0, sem1):
        wid = lax.axis_index("s") * NC + lax.axis_index("c")
        nt = bpw // _GC
        sems = (sem0, sem1)
        pltpu.sync_copy(ridx_hbm.at[pl.ds(wid * bpw, bpw)], idx_v)

        def start(t, b):
            return pltpu.async_copy(
                xf_hbm.at[idx_v.at[pl.ds(t * _GC, _GC)]], rows_v.at[b],
                sems[b])

        cps = [None, None]
        cps[0] = start(0, 0)
        for t in range(nt):
            b = t % 2
            if t + 1 < nt:
                cps[1 - b] = start(t + 1, 1 - b)
            cps[b].wait()
            pltpu.sync_copy(rows_v.at[b],
                            xg_hbm.at[pl.ds(wid * bpw + t * _GC, _GC)])

    return g


_gather_half = _make_gather(N // 2)


def _ffn_body(xg_ref, w1_ref, b1_ref, w2_ref, b2_ref, v_ref, out_ref):
    x = xg_ref[0].astype(jnp.bfloat16)
    w1 = w1_ref[0].astype(jnp.bfloat16)
    h = jnp.dot(x, w1, preferred_element_type=jnp.float32) + b1_ref[0]
    g = jax.nn.gelu(h, approximate=True).astype(jnp.bfloat16)
    w2 = w2_ref[0].astype(jnp.bfloat16)
    y = jnp.dot(g, w2, preferred_element_type=jnp.float32) + b2_ref[0]
    out_ref[...] = (y * v_ref[0])[None]


def _ffn_group(xg_half, w1, b1, w2, b2, v3, off):
    ge = E // 2
    return pl.pallas_call(
        _ffn_body,
        grid=(ge,),
        in_specs=[
            pl.BlockSpec((1, K, D), lambda i: (i, 0, 0)),
            pl.BlockSpec((1, D, DFF), lambda i: (i + off, 0, 0)),
            pl.BlockSpec((1, 1, DFF), lambda i: (i + off, 0, 0)),
            pl.BlockSpec((1, DFF, D), lambda i: (i + off, 0, 0)),
            pl.BlockSpec((1, 1, D), lambda i: (i + off, 0, 0)),
            pl.BlockSpec((1, K, 1), lambda i: (i + off, 0, 0)),
        ],
        out_specs=pl.BlockSpec((1, K, D), lambda i: (i, 0, 0)),
        out_shape=jax.ShapeDtypeStruct((ge, K, D), jnp.float32),
    )(xg_half, w1, b1, w2, b2, v3)


_Q = 8
_QC = D // _Q
_SC = 128


@functools.partial(
    pl.kernel,
    out_type=jax.ShapeDtypeStruct((N, D), jnp.float32),
    mesh=plsc.VectorSubcoreMesh(**_MESH),
    compiler_params=pltpu.CompilerParams(needs_layout_passes=False),
    scratch_types=[
        pltpu.VMEM((_SC, _QC), jnp.float32),
        pltpu.VMEM((N // NS // _SC, _SC), jnp.int32),
        pltpu.VMEM((2, _SC, _QC), jnp.float32),
        pltpu.VMEM_SHARED((N, _QC), jnp.float32),
        pltpu.SemaphoreType.DMA,
        pltpu.SemaphoreType.DMA,
    ],
)
def _scatter(yg1_hbm, yg2_hbm, ridx_hbm, zeros_hbm, out_hbm, zbuf, idx_v,
             ygbuf, acc, sem0, sem1):
    c = lax.axis_index("c")
    sid = lax.axis_index("s")
    rpw = N // NS
    nt = rpw // _SC
    wbase = sid * rpw
    hrow = sid - NS // 2
    sems = (sem0, sem1)
    pltpu.sync_copy(zeros_hbm, zbuf)
    for t in range(nt):
        pltpu.sync_copy(ridx_hbm.at[pl.ds(wbase + t * _SC, _SC)], idx_v.at[t])
    for j in range(_Q // NC):
        q = (c * (_Q // NC) + j) * _QC

        def start(src, rb, t, b):
            return pltpu.async_copy(
                src.at[pl.ds(rb + t * _SC, _SC), pl.ds(q, _QC)],
                ygbuf.at[b], sems[b])

        for i in range(nt):
            pltpu.sync_copy(zbuf, acc.at[pl.ds(wbase + i * _SC, _SC)])
        plsc.subcore_barrier()

        def accumulate(src, rb):
            cps = [None, None]
            cps[0] = start(src, rb, 0, 0)
            for t in range(nt):
                b = t % 2
                if t + 1 < nt:
                    cps[1 - b] = start(src, rb, t + 1, 1 - b)
                cps[b].wait()
                pltpu.sync_copy(ygbuf.at[b], acc.at[idx_v.at[t]], add=True)

        @pl.when(sid < NS // 2)
        def _():
            accumulate(yg1_hbm, sid * rpw)

        @pl.when(sid >= NS // 2)
        def _():
            accumulate(yg2_hbm, hrow * rpw)

        plsc.subcore_barrier()
        pltpu.sync_copy(acc.at[pl.ds(wbase, rpw)],
                        out_hbm.at[pl.ds(wbase, rpw), pl.ds(q, _QC)])
        plsc.subcore_barrier()


def kernel(x, W_switch, b_switch, W1, b1, W2, b2):
    Bx, Sx, Dx = x.shape
    xf = x.reshape(N, D)
    probs_t = _router(xf, W_switch, b_switch.reshape(E, 1))
    tau128, bud128 = _thresh(probs_t)
    routes, vals = _compact(probs_t, tau128, bud128)
    ridx = routes.reshape(N)
    xg1 = _gather_half(xf, ridx[: N // 2])
    xg2 = _gather_half(xf, ridx[N // 2:])
    b1r = b1.reshape(E, 1, DFF)
    b2r = b2.reshape(E, 1, D)
    v3 = vals[:, :, None]
    yg1 = _ffn_group(xg1.reshape(E // 2, K, D), W1, b1r, W2, b2r, v3, 0)
    yg2 = _ffn_group(xg2.reshape(E // 2, K, D), W1, b1r, W2, b2r, v3, E // 2)
    zeros = jnp.zeros((_SC, _QC), jnp.float32)
    out = _scatter(yg1.reshape(N // 2, D), yg2.reshape(N // 2, D), ridx, zeros)
    return out.reshape(Bx, Sx, Dx)


# speedup vs baseline: 1.0183x; 1.0183x over previous
import functools

import jax
import jax.numpy as jnp
from jax import lax
from jax.experimental import pallas as pl
from jax.experimental.pallas import tpu as pltpu
from jax.experimental.pallas import tpu_sc as plsc

E = 16
D = 1024
DFF = 1024
N = 8192
K = N // E
L = 16
NC, NS = 2, 16
NW = NC * NS

_MESH = dict(core_axis_name="c", subcore_axis_name="s", num_cores=NC,
             num_subcores=NS)


_TB = 1024


def _router_body(x_ref, w_ref, b_ref, out_ref):
    lg = lax.dot_general(w_ref[...], x_ref[...],
                         (((0,), (1,)), ((), ())),
                         preferred_element_type=jnp.float32)
    lg = lg + b_ref[...]
    m = jnp.max(lg, axis=0, keepdims=True)
    ex = jnp.exp(lg - m)
    out_ref[...] = ex / jnp.sum(ex, axis=0, keepdims=True)


def _router(xf, w, b2d):
    return pl.pallas_call(
        _router_body,
        grid=(N // _TB,),
        in_specs=[
            pl.BlockSpec((_TB, D), lambda i: (i, 0)),
            pl.BlockSpec((D, E), lambda i: (0, 0)),
            pl.BlockSpec((E, 1), lambda i: (0, 0)),
        ],
        out_specs=pl.BlockSpec((E, _TB), lambda i: (0, i)),
        out_shape=jax.ShapeDtypeStruct((E, N), jnp.float32),
    )(xf, w, b2d)


def _thresh_body(pt_ref, tau_ref, bud_ref):
    bits = lax.bitcast_convert_type(pt_ref[...], jnp.int32)

    def body(_, carry):
        lo, hi = carry
        mid = lo + lax.div(hi - lo + 1, 2)
        cnt = jnp.sum((bits >= mid).astype(jnp.int32), axis=1, keepdims=True)
        ge = cnt >= K
        return jnp.where(ge, mid, lo), jnp.where(ge, hi, mid - 1)

    lo0 = jnp.zeros((E, 1), jnp.int32)
    hi0 = jnp.full((E, 1), jnp.int32(0x3F800001))
    lo, _ = lax.fori_loop(0, 31, body, (lo0, hi0))
    cnt_gt = jnp.sum((bits > lo).astype(jnp.int32), axis=1, keepdims=True)
    tau_ref[...] = jnp.broadcast_to(lo, (E, 128))
    bud_ref[...] = jnp.broadcast_to(K - cnt_gt, (E, 128))


def _thresh(probs_t):
    return pl.pallas_call(
        _thresh_body,
        out_shape=(jax.ShapeDtypeStruct((E, 128), jnp.int32),
                   jax.ShapeDtypeStruct((E, 128), jnp.int32)),
    )(probs_t)


@functools.partial(
    pl.kernel,
    out_type=(jax.ShapeDtypeStruct((E, K), jnp.int32),
              jax.ShapeDtypeStruct((E, K), jnp.float32)),
    mesh=plsc.VectorSubcoreMesh(**_MESH),
    compiler_params=pltpu.CompilerParams(needs_layout_passes=False),
    scratch_types=[
        pltpu.VMEM((N,), jnp.float32),
        pltpu.VMEM((K,), jnp.int32),
        pltpu.VMEM((K,), jnp.float32),
        pltpu.VMEM((L,), jnp.int32),
        pltpu.VMEM((L,), jnp.int32),
    ],
)
def _compact(pt_hbm, tau_hbm, bud_hbm, routes_hbm, vals_hbm,
             row_v, idx_v, val_v, tau_v, bud_v):
    wid = lax.axis_index("s") * NC + lax.axis_index("c")

    @pl.when(wid < E)
    def _():
        e = wid
        pltpu.sync_copy(pt_hbm.at[e], row_v)
        pltpu.sync_copy(tau_hbm.at[e, pl.ds(0, L)], tau_v)
        pltpu.sync_copy(bud_hbm.at[e, pl.ds(0, L)], bud_v)
        lane = lax.iota(jnp.int32, L)
        tau_vec = tau_v[...]
        bud_vec0 = bud_v[...]

        def body(j, carry):
            ptr, bud = carry
            p16 = row_v[pl.ds(j * L, L)]
            bits = plsc.bitcast(p16, jnp.int32)
            gt = bits > tau_vec
            eq = bits == tau_vec
            eqc = plsc.cumsum(jnp.where(eq, 1, 0))
            take_eq = eq & (eqc <= bud)
            mask = gt | take_eq
            pos = ptr + plsc.cumsum(jnp.where(mask, 1, 0)) - 1
            ids = j * L + lane
            plsc.store_scatter(idx_v, [pos], ids, mask=mask)
            plsc.store_scatter(val_v, [pos], p16, mask=mask)
            cnt = plsc.all_reduce_population_count(mask)
            ceq = plsc.all_reduce_population_count(take_eq)
            return ptr + cnt, bud - ceq

        zero16 = jnp.zeros((L,), jnp.int32)
        lax.fori_loop(0, N // L, body, (zero16, bud_vec0))
        pltpu.sync_copy(idx_v, routes_hbm.at[e])
        pltpu.sync_copy(val_v, vals_hbm.at[e])


_GC = 32


def _make_gather(rows):
    bpw = rows // NW

    @functools.partial(
        pl.kernel,
        out_type=jax.ShapeDtypeStruct((rows, D), jnp.float32),
        mesh=plsc.VectorSubcoreMesh(**_MESH),
        compiler_params=pltpu.CompilerParams(needs_layout_passes=False),
        scratch_types=[
            pltpu.VMEM((bpw,), jnp.int32),
            pltpu.VMEM((2, _GC, D), jnp.float32),
            pltpu.SemaphoreType.DMA,
            pltpu.SemaphoreType.DMA,
        ],
    )
    def g(xf_hbm, ridx_hbm, xg_hbm, idx_v, rows_v, sem0, sem1):
        wid = lax.axis_index("s") * NC + lax.axis_index("c")
        nt = bpw // _GC
        sems = (sem0, sem1)
        pltpu.sync_copy(ridx_hbm.at[pl.ds(wid * bpw, bpw)], idx_v)

        def start(t, b):
            return pltpu.async_copy(
                xf_hbm.at[idx_v.at[pl.ds(t * _GC, _GC)]], rows_v.at[b],
                sems[b])

        cps = [None, None]
        cps[0] = start(0, 0)
        for t in range(nt):
            b = t % 2
            if t + 1 < nt:
                cps[1 - b] = start(t + 1, 1 - b)
            cps[b].wait()
            pltpu.sync_copy(rows_v.at[b],
                            xg_hbm.at[pl.ds(wid * bpw + t * _GC, _GC)])

    return g


_gather_all = _make_gather(N)


def _ffn_body(xg_ref, w1_ref, b1_ref, w2_ref, b2_ref, v_ref, out_ref):
    x = xg_ref[0].astype(jnp.bfloat16)
    w1 = w1_ref[0].astype(jnp.bfloat16)
    h = jnp.dot(x, w1, preferred_element_type=jnp.float32) + b1_ref[0]
    g = jax.nn.gelu(h, approximate=True).astype(jnp.bfloat16)
    w2 = w2_ref[0].astype(jnp.bfloat16)
    y = jnp.dot(g, w2, preferred_element_type=jnp.float32) + b2_ref[0]
    out_ref[...] = (y * v_ref[0])[None]


def _ffn(xg, w1, b1, w2, b2, v3):
    return pl.pallas_call(
        _ffn_body,
        grid=(E,),
        in_specs=[
            pl.BlockSpec((1, K, D), lambda i: (i, 0, 0)),
            pl.BlockSpec((1, D, DFF), lambda i: (i, 0, 0)),
            pl.BlockSpec((1, 1, DFF), lambda i: (i, 0, 0)),
            pl.BlockSpec((1, DFF, D), lambda i: (i, 0, 0)),
            pl.BlockSpec((1, 1, D), lambda i: (i, 0, 0)),
            pl.BlockSpec((1, K, 1), lambda i: (i, 0, 0)),
        ],
        out_specs=pl.BlockSpec((1, K, D), lambda i: (i, 0, 0)),
        out_shape=jax.ShapeDtypeStruct((E, K, D), jnp.float32),
    )(xg, w1, b1, w2, b2, v3)


_Q = 8
_QC = D // _Q
_SC = 128


@functools.partial(
    pl.kernel,
    out_type=jax.ShapeDtypeStruct((N, D), jnp.float32),
    mesh=plsc.VectorSubcoreMesh(**_MESH),
    compiler_params=pltpu.CompilerParams(needs_layout_passes=False),
    scratch_types=[
        pltpu.VMEM((_SC, _QC), jnp.float32),
        pltpu.VMEM((N // NS // _SC, _SC), jnp.int32),
        pltpu.VMEM((2, _SC, _QC), jnp.float32),
        pltpu.VMEM_SHARED((N, _QC), jnp.float32),
        pltpu.SemaphoreType.DMA,
        pltpu.SemaphoreType.DMA,
    ],
)
def _scatter(yg_hbm, ridx_hbm, zeros_hbm, out_hbm, zbuf, idx_v,
             ygbuf, acc, sem0, sem1):
    c = lax.axis_index("c")
    sid = lax.axis_index("s")
    rpw = N // NS
    nt = rpw // _SC
    wbase = sid * rpw
    sems = (sem0, sem1)
    pltpu.sync_copy(zeros_hbm, zbuf)
    for t in range(nt):
        pltpu.sync_copy(ridx_hbm.at[pl.ds(wbase + t * _SC, _SC)], idx_v.at[t])
    for j in range(_Q // NC):
        q = (c * (_Q // NC) + j) * _QC

        def start(t, b):
            return pltpu.async_copy(
                yg_hbm.at[pl.ds(wbase + t * _SC, _SC), pl.ds(q, _QC)],
                ygbuf.at[b], sems[b])

        for i in range(nt):
            pltpu.sync_copy(zbuf, acc.at[pl.ds(wbase + i * _SC, _SC)])
        plsc.subcore_barrier()
        cps = [None, None]
        cps[0] = start(0, 0)
        for t in range(nt):
            b = t % 2
            if t + 1 < nt:
                cps[1 - b] = start(t + 1, 1 - b)
            cps[b].wait()
            pltpu.sync_copy(ygbuf.at[b], acc.at[idx_v.at[t]], add=True)
        plsc.subcore_barrier()
        pltpu.sync_copy(acc.at[pl.ds(wbase, rpw)],
                        out_hbm.at[pl.ds(wbase, rpw), pl.ds(q, _QC)])
        plsc.subcore_barrier()


def kernel(x, W_switch, b_switch, W1, b1, W2, b2):
    Bx, Sx, Dx = x.shape
    xf = x.reshape(N, D)
    probs_t = _router(xf, W_switch, b_switch.reshape(E, 1))
    tau128, bud128 = _thresh(probs_t)
    routes, vals = _compact(probs_t, tau128, bud128)
    ridx = routes.reshape(N)
    xg = _gather_all(xf, ridx)
    yg = _ffn(xg.reshape(E, K, D), W1, b1.reshape(E, 1, DFF), W2,
              b2.reshape(E, 1, D), vals[:, :, None])
    zeros = jnp.zeros((_SC, _QC), jnp.float32)
    out = _scatter(yg.reshape(N, D), ridx, zeros)
    return out.reshape(Bx, Sx, Dx)


# speedup vs baseline: 1.0259x; 1.0075x over previous
import functools

import jax
import jax.numpy as jnp
from jax import lax
from jax.experimental import pallas as pl
from jax.experimental.pallas import tpu as pltpu
from jax.experimental.pallas import tpu_sc as plsc

E = 16
D = 1024
DFF = 1024
N = 8192
K = N // E
L = 16
NC, NS = 2, 16
NW = NC * NS

_MESH = dict(core_axis_name="c", subcore_axis_name="s", num_cores=NC,
             num_subcores=NS)


_TB = 1024


def _router_body(x_ref, w_ref, b_ref, out_ref):
    lg = lax.dot_general(w_ref[...], x_ref[...],
                         (((0,), (1,)), ((), ())),
                         preferred_element_type=jnp.float32)
    lg = lg + b_ref[...]
    m = jnp.max(lg, axis=0, keepdims=True)
    ex = jnp.exp(lg - m)
    out_ref[...] = ex / jnp.sum(ex, axis=0, keepdims=True)


def _router(xf, w, b2d):
    return pl.pallas_call(
        _router_body,
        grid=(N // _TB,),
        in_specs=[
            pl.BlockSpec((_TB, D), lambda i: (i, 0)),
            pl.BlockSpec((D, E), lambda i: (0, 0)),
            pl.BlockSpec((E, 1), lambda i: (0, 0)),
        ],
        out_specs=pl.BlockSpec((E, _TB), lambda i: (0, i)),
        out_shape=jax.ShapeDtypeStruct((E, N), jnp.float32),
    )(xf, w, b2d)


def _thresh_body(pt_ref, tau_ref, bud_ref):
    bits = lax.bitcast_convert_type(pt_ref[...], jnp.int32)

    def body(_, carry):
        lo, hi = carry
        mid = lo + lax.div(hi - lo + 1, 2)
        cnt = jnp.sum((bits >= mid).astype(jnp.int32), axis=1, keepdims=True)
        ge = cnt >= K
        return jnp.where(ge, mid, lo), jnp.where(ge, hi, mid - 1)

    lo0 = jnp.zeros((E, 1), jnp.int32)
    hi0 = jnp.full((E, 1), jnp.int32(0x3F800001))
    lo, _ = lax.fori_loop(0, 31, body, (lo0, hi0))
    cnt_gt = jnp.sum((bits > lo).astype(jnp.int32), axis=1, keepdims=True)
    tau_ref[...] = jnp.broadcast_to(lo, (E, 128))
    bud_ref[...] = jnp.broadcast_to(K - cnt_gt, (E, 128))


def _thresh(probs_t):
    return pl.pallas_call(
        _thresh_body,
        out_shape=(jax.ShapeDtypeStruct((E, 128), jnp.int32),
                   jax.ShapeDtypeStruct((E, 128), jnp.int32)),
    )(probs_t)


@functools.partial(
    pl.kernel,
    out_type=(jax.ShapeDtypeStruct((E, K), jnp.int32),
              jax.ShapeDtypeStruct((E, K), jnp.float32)),
    mesh=plsc.VectorSubcoreMesh(**_MESH),
    compiler_params=pltpu.CompilerParams(needs_layout_passes=False),
    scratch_types=[
        pltpu.VMEM((N,), jnp.float32),
        pltpu.VMEM((K,), jnp.int32),
        pltpu.VMEM((K,), jnp.float32),
        pltpu.VMEM((L,), jnp.int32),
        pltpu.VMEM((L,), jnp.int32),
    ],
)
def _compact(pt_hbm, tau_hbm, bud_hbm, routes_hbm, vals_hbm,
             row_v, idx_v, val_v, tau_v, bud_v):
    wid = lax.axis_index("s") * NC + lax.axis_index("c")

    @pl.when(wid < E)
    def _():
        e = wid
        pltpu.sync_copy(pt_hbm.at[e], row_v)
        pltpu.sync_copy(tau_hbm.at[e, pl.ds(0, L)], tau_v)
        pltpu.sync_copy(bud_hbm.at[e, pl.ds(0, L)], bud_v)
        lane = lax.iota(jnp.int32, L)
        tau_vec = tau_v[...]
        bud_vec0 = bud_v[...]

        def body(j, carry):
            ptr, bud = carry
            p16 = row_v[pl.ds(j * L, L)]
            bits = plsc.bitcast(p16, jnp.int32)
            gt = bits > tau_vec
            eq = bits == tau_vec
            eqc = plsc.cumsum(jnp.where(eq, 1, 0))
            take_eq = eq & (eqc <= bud)
            mask = gt | take_eq
            pos = ptr + plsc.cumsum(jnp.where(mask, 1, 0)) - 1
            ids = j * L + lane
            plsc.store_scatter(idx_v, [pos], ids, mask=mask)
            plsc.store_scatter(val_v, [pos], p16, mask=mask)
            cnt = plsc.all_reduce_population_count(mask)
            ceq = plsc.all_reduce_population_count(take_eq)
            return ptr + cnt, bud - ceq

        zero16 = jnp.zeros((L,), jnp.int32)
        lax.fori_loop(0, N // L, body, (zero16, bud_vec0))
        pltpu.sync_copy(idx_v, routes_hbm.at[e])
        pltpu.sync_copy(val_v, vals_hbm.at[e])


_GC = 32


def _make_gather(rows):
    bpw = rows // NW

    @functools.partial(
        pl.kernel,
        out_type=jax.ShapeDtypeStruct((rows, D), jnp.float32),
        mesh=plsc.VectorSubcoreMesh(**_MESH),
        compiler_params=pltpu.CompilerParams(needs_layout_passes=False),
        scratch_types=[
            pltpu.VMEM((bpw,), jnp.int32),
            pltpu.VMEM((2, _GC, D), jnp.float32),
            pltpu.SemaphoreType.DMA,
            pltpu.SemaphoreType.DMA,
        ],
    )
    def g(xf_hbm, ridx_hbm, xg_hbm, idx_v, rows_v, sem0, sem1):
        wid = lax.axis_index("s") * NC + lax.axis_index("c")
        nt = bpw // _GC
        sems = (sem0, sem1)
        pltpu.sync_copy(ridx_hbm.at[pl.ds(wid * bpw, bpw)], idx_v)

        def start(t, b):
            return pltpu.async_copy(
                xf_hbm.at[idx_v.at[pl.ds(t * _GC, _GC)]], rows_v.at[b],
                sems[b])

        cps = [None, None]
        cps[0] = start(0, 0)
        for t in range(nt):
            b = t % 2
            if t + 1 < nt:
                cps[1 - b] = start(t + 1, 1 - b)
            cps[b].wait()
            pltpu.sync_copy(rows_v.at[b],
                            xg_hbm.at[pl.ds(wid * bpw + t * _GC, _GC)])

    return g


_gather_all = _make_gather(N)


def _ffn_body(xg_ref, w1_ref, b1_ref, w2_ref, b2_ref, v_ref, out_ref):
    x = xg_ref[0].astype(jnp.bfloat16)
    w1 = w1_ref[0].astype(jnp.bfloat16)
    h = jnp.dot(x, w1, preferred_element_type=jnp.float32) + b1_ref[0]
    g = jax.nn.gelu(h, approximate=True).astype(jnp.bfloat16)
    w2 = w2_ref[0].astype(jnp.bfloat16)
    y = jnp.dot(g, w2, preferred_element_type=jnp.float32) + b2_ref[0]
    out_ref[...] = (y * v_ref[0])[None]


def _ffn(xg, w1, b1, w2, b2, v3):
    return pl.pallas_call(
        _ffn_body,
        grid=(E,),
        in_specs=[
            pl.BlockSpec((1, K, D), lambda i: (i, 0, 0)),
            pl.BlockSpec((1, D, DFF), lambda i: (i, 0, 0)),
            pl.BlockSpec((1, 1, DFF), lambda i: (i, 0, 0)),
            pl.BlockSpec((1, DFF, D), lambda i: (i, 0, 0)),
            pl.BlockSpec((1, 1, D), lambda i: (i, 0, 0)),
            pl.BlockSpec((1, K, 1), lambda i: (i, 0, 0)),
        ],
        out_specs=pl.BlockSpec((1, K, D), lambda i: (i, 0, 0)),
        out_shape=jax.ShapeDtypeStruct((E, K, D), jnp.float32),
    )(xg, w1, b1, w2, b2, v3)


_Q = 8
_QC = D // _Q
_SC = 128


@functools.partial(
    pl.kernel,
    out_type=jax.ShapeDtypeStruct((N, D), jnp.float32),
    mesh=plsc.VectorSubcoreMesh(**_MESH),
    compiler_params=pltpu.CompilerParams(needs_layout_passes=False),
    scratch_types=[
        pltpu.VMEM((_SC, _QC), jnp.float32),
        pltpu.VMEM((N // NS // _SC, _SC), jnp.int32),
        pltpu.VMEM((2, _SC, _QC), jnp.float32),
        pltpu.VMEM_SHARED((N, _QC), jnp.float32),
        pltpu.SemaphoreType.DMA,
        pltpu.SemaphoreType.DMA,
    ],
)
def _scatter(yg_hbm, ridx_hbm, zeros_hbm, out_hbm, zbuf, idx_v,
             ygbuf, acc, sem0, sem1):
    c = lax.axis_index("c")
    sid = lax.axis_index("s")
    rpw = N // NS
    nt = rpw // _SC
    wbase = sid * rpw
    sems = (sem0, sem1)
    pltpu.sync_copy(zeros_hbm, zbuf)
    for t in range(nt):
        pltpu.sync_copy(ridx_hbm.at[pl.ds(wbase + t * _SC, _SC)], idx_v.at[t])
    for i in range(nt):
        pltpu.sync_copy(zbuf, acc.at[pl.ds(wbase + i * _SC, _SC)])
    plsc.subcore_barrier()
    nsl = _Q // NC
    for j in range(nsl):
        q = (c * nsl + j) * _QC

        def start(t, b):
            return pltpu.async_copy(
                yg_hbm.at[pl.ds(wbase + t * _SC, _SC), pl.ds(q, _QC)],
                ygbuf.at[b], sems[b])

        cps = [None, None]
        cps[0] = start(0, 0)
        for t in range(nt):
            b = t % 2
            if t + 1 < nt:
                cps[1 - b] = start(t + 1, 1 - b)
            cps[b].wait()
            pltpu.sync_copy(ygbuf.at[b], acc.at[idx_v.at[t]], add=True)
        plsc.subcore_barrier()
        pltpu.sync_copy(acc.at[pl.ds(wbase, rpw)],
                        out_hbm.at[pl.ds(wbase, rpw), pl.ds(q, _QC)])
        if j + 1 < nsl:
            for i in range(nt):
                pltpu.sync_copy(zbuf, acc.at[pl.ds(wbase + i * _SC, _SC)])
            plsc.subcore_barrier()


def kernel(x, W_switch, b_switch, W1, b1, W2, b2):
    Bx, Sx, Dx = x.shape
    xf = x.reshape(N, D)
    probs_t = _router(xf, W_switch, b_switch.reshape(E, 1))
    tau128, bud128 = _thresh(probs_t)
    routes, vals = _compact(probs_t, tau128, bud128)
    ridx = routes.reshape(N)
    xg = _gather_all(xf, ridx)
    yg = _ffn(xg.reshape(E, K, D), W1, b1.reshape(E, 1, DFF), W2,
              b2.reshape(E, 1, D), vals[:, :, None])
    zeros = jnp.zeros((_SC, _QC), jnp.float32)
    out = _scatter(yg.reshape(N, D), ridx, zeros)
    return out.reshape(Bx, Sx, Dx)
